# Initial kernel scaffold; baseline (speedup 1.0000x reference)
import jax
import jax.numpy as jnp
from jax.experimental import pallas as pl


def kernel(bbs, conf):
    raise NotImplementedError("write your pallas kernel here")


# speedup vs baseline: 52.6711x; 52.6711x over previous
import functools

import jax
import jax.numpy as jnp
from jax import lax
from jax.experimental import pallas as pl
from jax.experimental.pallas import tpu as pltpu
from jax.experimental.pallas import tpu_sc as plsc

NMS_THR = 0.45
NT = 16
L = 16


def _tec_body(x1h, y1h, x2h, y2h, areah, confh, outh,
              vx1, vy1, vx2, vy2, varea, outs, act, candv, frontv, cand_sp,
              *, n, npad, sl):
    t = lax.axis_index("s")
    base = t * sl
    sent = jnp.int32(npad)

    pltpu.sync_copy(x1h, vx1)
    pltpu.sync_copy(y1h, vy1)
    pltpu.sync_copy(x2h, vx2)
    pltpu.sync_copy(y2h, vy2)
    pltpu.sync_copy(areah, varea)
    pltpu.sync_copy(confh.at[pl.ds(base, sl)], outs)

    cnt = jnp.minimum(jnp.maximum(jnp.int32(n) - base, 0), jnp.int32(sl))

    def init_chunk(c, _):
        vals = base + c * L + lax.iota(jnp.int32, L)
        act[pl.ds(c * L, L)] = jnp.minimum(vals, jnp.int32(npad - 1))
        return 0

    lax.fori_loop(0, (sl + L) // L, init_chunk, 0)

    lane = lax.iota(jnp.int32, L)

    def publish_and_min(my_cnt):
        v = act[pl.ds(0, L)]
        front = jnp.where(my_cnt > 0, v[0], sent)
        frontv[...] = jnp.full((L,), front, jnp.int32)
        pltpu.sync_copy(frontv, cand_sp.at[pl.ds(t * L, L)])
        plsc.subcore_barrier()
        pltpu.sync_copy(cand_sp, candv)
        m = candv[pl.ds(0, L)]
        for r in range(1, NT):
            m = jnp.minimum(m, candv[pl.ds(r * L, L)])
        plsc.subcore_barrier()
        return m[0]

    def body(carry):
        m, my_cnt = carry
        mvec = jnp.full((L,), m, jnp.int32)
        bx1 = plsc.load_gather(vx1, [mvec])
        by1 = plsc.load_gather(vy1, [mvec])
        bx2 = plsc.load_gather(vx2, [mvec])
        by2 = plsc.load_gather(vy2, [mvec])
        ba = plsc.load_gather(varea, [mvec])
        nchunks = (my_cnt + (L - 1)) // L

        def chunk(c, w):
            idxv = act[pl.ds(c * L, L)]
            valid = (c * L + lane) < my_cnt
            gx1 = plsc.load_gather(vx1, [idxv], mask=valid)
            gy1 = plsc.load_gather(vy1, [idxv], mask=valid)
            gx2 = plsc.load_gather(vx2, [idxv], mask=valid)
            gy2 = plsc.load_gather(vy2, [idxv], mask=valid)
            ga = plsc.load_gather(varea, [idxv], mask=valid)
            xx1 = jnp.maximum(gx1, bx1)
            yy1 = jnp.maximum(gy1, by1)
            xx2 = jnp.minimum(gx2, bx2)
            yy2 = jnp.minimum(gy2, by2)
            inter = (jnp.maximum(xx2 - xx1, 0.0)
                     * jnp.maximum(yy2 - yy1, 0.0))
            iou = inter / (ba + ga - inter + 1e-12)
            not_self = idxv != mvec
            supp = (iou > NMS_THR) & not_self & valid
            keep = valid & not_self & jnp.logical_not(supp)
            zl = jnp.clip(idxv - base, 0, sl - 1)
            plsc.store_scatter(outs, [zl], jnp.zeros((L,), jnp.float32),
                               mask=supp)
            plsc.store_compressed(act.at[pl.ds(w, L)], idxv, mask=keep)
            return w + plsc.all_reduce_population_count(keep)[0]

        new_cnt = lax.fori_loop(0, nchunks, chunk, jnp.int32(0))
        return publish_and_min(new_cnt), new_cnt

    def round_body(i, carry):
        return lax.cond(carry[0] < sent, body, lambda c: c, carry)

    m0 = publish_and_min(cnt)
    lax.fori_loop(0, n, round_body, (m0, cnt))

    pltpu.sync_copy(outs, outh.at[pl.ds(base, sl)])


@functools.lru_cache(maxsize=None)
def _build(n):
    sl = -(-n // (NT * L * 8)) * (L * 8)
    npad = NT * sl
    mesh = plsc.VectorSubcoreMesh(core_axis_name="c", subcore_axis_name="s",
                                  num_cores=1, num_subcores=NT)
    body = functools.partial(_tec_body, n=n, npad=npad, sl=sl)
    call = pl.kernel(
        body,
        out_type=jax.ShapeDtypeStruct((npad,), jnp.float32),
        mesh=mesh,
        compiler_params=pltpu.CompilerParams(needs_layout_passes=False),
        scratch_types=[
            pltpu.VMEM((npad,), jnp.float32),
            pltpu.VMEM((npad,), jnp.float32),
            pltpu.VMEM((npad,), jnp.float32),
            pltpu.VMEM((npad,), jnp.float32),
            pltpu.VMEM((npad,), jnp.float32),
            pltpu.VMEM((sl,), jnp.float32),
            pltpu.VMEM((sl + L,), jnp.int32),
            pltpu.VMEM((NT * L,), jnp.int32),
            pltpu.VMEM((L,), jnp.int32),
            pltpu.VMEM_SHARED((NT * L,), jnp.int32),
        ],
    )
    return call, npad


def kernel(bbs, conf):
    n = conf.shape[0]
    call, npad = _build(n)
    order = jnp.argsort(-conf)
    sb = bbs[order]
    sconf = conf[order]
    x1 = sb[:, 0]
    y1 = sb[:, 1]
    x2 = sb[:, 0] + sb[:, 2]
    y2 = sb[:, 1] + sb[:, 3]
    area = jnp.maximum(x2 - x1, 0.0) * jnp.maximum(y2 - y1, 0.0)
    pad = npad - n

    def p(a):
        return jnp.concatenate([a, jnp.zeros((pad,), a.dtype)])

    out_sorted = call(p(x1), p(y1), p(x2), p(y2), p(area), p(sconf))
    return jnp.zeros_like(conf).at[order].set(out_sorted[:n])


# speedup vs baseline: 98.0140x; 1.8609x over previous
import functools

import jax
import jax.numpy as jnp
from jax import lax
from jax.experimental import pallas as pl
from jax.experimental.pallas import tpu as pltpu
from jax.experimental.pallas import tpu_sc as plsc

NMS_THR = 0.45
NT = 16
L = 16
CB = 64

def _sort16(v, lane):
    for k in (2, 4, 8, 16):
        j = k // 2
        while j >= 1:
            pv = v[jnp.bitwise_xor(lane, j)]
            takemin = ((lane & j) == 0) == ((lane & k) == 0)
            v = jnp.where(takemin, jnp.minimum(v, pv), jnp.maximum(v, pv))
            j //= 2
    return v


def _tec_body(x1h, y1h, x2h, y2h, confh, outh,
              vx1, vy1, vx2, vy2, outs, act, pubv, candv, cand_sp,
              *, n, npad, sl):
    t = lax.axis_index("s")
    sent = jnp.int32(npad)
    lane = lax.iota(jnp.int32, L)
    zf = jnp.zeros((L,), jnp.float32)

    pltpu.sync_copy(x1h, vx1)
    pltpu.sync_copy(y1h, vy1)
    pltpu.sync_copy(x2h, vx2)
    pltpu.sync_copy(y2h, vy2)
    pltpu.sync_copy(confh.at[t], outs)

    def init_chunk(c, _):
        act[pl.ds(c * L, L)] = jnp.minimum((c * L + lane) * NT + t, sent - 1)
        return 0

    lax.fori_loop(0, sl // L, init_chunk, 0)
    cnt0 = (jnp.maximum(jnp.int32(n) - t, 0) + (NT - 1)) // NT

    def round_fn(cnt, par):
        v0 = act[pl.ds(0, L)]
        front = jnp.where(cnt > 0, v0[0], sent)
        second = jnp.where(cnt > 1, v0[1], sent)
        pubv[pl.ds(0, L)] = jnp.full((L,), front, jnp.int32)
        pubv[pl.ds(L, L)] = jnp.full((L,), second, jnp.int32)
        pltpu.sync_copy(pubv,
                        cand_sp.at[pl.ds((par * NT + t) * (2 * L), 2 * L)])
        plsc.subcore_barrier()
        pltpu.sync_copy(cand_sp.at[pl.ds(par * NT * 2 * L, NT * 2 * L)],
                        candv)

        fronts = jnp.full((L,), sent, jnp.int32)
        seconds = jnp.full((L,), sent, jnp.int32)
        for r in range(NT):
            sel = lane == r
            fronts = jnp.where(sel, candv[pl.ds(r * 2 * L, L)], fronts)
            seconds = jnp.where(sel, candv[pl.ds(r * 2 * L + L, L)], seconds)

        m = seconds
        for d in (8, 4, 2, 1):
            m = jnp.minimum(m, m[jnp.bitwise_xor(lane, d)])
        lim = m[0]
        limv = jnp.full((L,), lim, jnp.int32)

        cand = _sort16(fronts, lane)
        gdone = cand[0] >= sent
        cg = jnp.minimum(cand, sent - 1)
        cx1 = plsc.load_gather(vx1, [cg])
        cy1 = plsc.load_gather(vy1, [cg])
        cx2 = plsc.load_gather(vx2, [cg])
        cy2 = plsc.load_gather(vy2, [cg])
        carea = (jnp.maximum(cx2 - cx1, 0.0) * jnp.maximum(cy2 - cy1, 0.0))
        eligible = cand < limv
        supp = jnp.zeros((L,), jnp.bool_)
        for k in range(L):
            kf = jnp.full((L,), k, jnp.int32)
            state = (eligible & jnp.logical_not(supp)).astype(jnp.int32)
            kept_k = state[kf] != 0
            bx1 = cx1[kf]
            by1 = cy1[kf]
            bx2 = cx2[kf]
            by2 = cy2[kf]
            ba = carea[kf]
            xx1 = jnp.maximum(cx1, bx1)
            yy1 = jnp.maximum(cy1, by1)
            xx2 = jnp.minimum(cx2, bx2)
            yy2 = jnp.minimum(cy2, by2)
            inter = (jnp.maximum(xx2 - xx1, 0.0)
                     * jnp.maximum(yy2 - yy1, 0.0))
            iou = inter / (ba + carea - inter + 1e-12)
            supp = supp | ((iou > NMS_THR) & (lane > kf) & kept_k)
        keep = eligible & jnp.logical_not(supp)

        frontv = jnp.full((L,), front, jnp.int32)
        mydecided = front < lim
        mysupp_cnt = plsc.all_reduce_population_count(
            (cand == frontv) & supp & eligible)[0]
        myslot = jnp.minimum(front // NT, jnp.int32(sl - 1))
        plsc.store_scatter(outs, [jnp.full((L,), myslot, jnp.int32)], zf,
                           mask=(lane == 0) & mydecided & (mysupp_cnt > 0))

        kvi = keep.astype(jnp.int32)
        skip = jnp.where(mydecided, jnp.int32(1), jnp.int32(0))
        ncand = cnt - skip
        nchunks = (ncand + (L - 1)) // L

        def chunk(c, w):
            idxv = act[pl.ds(skip + c * L, L)]
            valid = (c * L + lane) < ncand
            gx1 = plsc.load_gather(vx1, [idxv], mask=valid)
            gy1 = plsc.load_gather(vy1, [idxv], mask=valid)
            gx2 = plsc.load_gather(vx2, [idxv], mask=valid)
            gy2 = plsc.load_gather(vy2, [idxv], mask=valid)
            ga = (jnp.maximum(gx2 - gx1, 0.0) * jnp.maximum(gy2 - gy1, 0.0))
            sacc = jnp.zeros((L,), jnp.bool_)
            for k in range(L):
                kf = jnp.full((L,), k, jnp.int32)
                kk = kvi[kf] != 0
                bx1 = cx1[kf]
                by1 = cy1[kf]
                bx2 = cx2[kf]
                by2 = cy2[kf]
                ba = carea[kf]
                xx1 = jnp.maximum(gx1, bx1)
                yy1 = jnp.maximum(gy1, by1)
                xx2 = jnp.minimum(gx2, bx2)
                yy2 = jnp.minimum(gy2, by2)
                inter = (jnp.maximum(xx2 - xx1, 0.0)
                         * jnp.maximum(yy2 - yy1, 0.0))
                iou = inter / (ba + ga - inter + 1e-12)
                sacc = sacc | ((iou > NMS_THR) & kk)
            sacc = sacc & valid
            keepv = valid & jnp.logical_not(sacc)
            plsc.store_scatter(outs, [idxv // NT], zf, mask=sacc)
            plsc.store_compressed(act.at[pl.ds(w, L)], idxv, mask=keepv)
            return w + plsc.all_reduce_population_count(keepv)[0]

        newcnt = lax.fori_loop(0, nchunks, chunk, jnp.int32(0))
        return newcnt, gdone.astype(jnp.int32)

    def block_fn(b, carry):
        def run_block(carry):
            def one_round(i, carry):
                cnt, done = carry
                return lax.cond(done == 1, lambda c: (c, jnp.int32(1)),
                                lambda c: round_fn(c, i % 2), cnt)
            return lax.fori_loop(0, CB, one_round, carry)

        return lax.cond(carry[1] == 1, lambda c: c, run_block, carry)

    nblocks = (n + 1 + CB - 1) // CB
    lax.fori_loop(0, nblocks, block_fn, (cnt0, jnp.int32(0)))

    pltpu.sync_copy(outs, outh.at[t])


@functools.lru_cache(maxsize=None)
def _build(n):
    npad = -(-n // (NT * L)) * (NT * L)
    sl = npad // NT
    mesh = plsc.VectorSubcoreMesh(core_axis_name="c", subcore_axis_name="s",
                                  num_cores=1, num_subcores=NT)
    body = functools.partial(_tec_body, n=n, npad=npad, sl=sl)
    call = pl.kernel(
        body,
        out_type=jax.ShapeDtypeStruct((NT, sl), jnp.float32),
        mesh=mesh,
        compiler_params=pltpu.CompilerParams(needs_layout_passes=False),
        scratch_types=[
            pltpu.VMEM((npad,), jnp.float32),
            pltpu.VMEM((npad,), jnp.float32),
            pltpu.VMEM((npad,), jnp.float32),
            pltpu.VMEM((npad,), jnp.float32),
            pltpu.VMEM((sl,), jnp.float32),
            pltpu.VMEM((sl + L,), jnp.int32),
            pltpu.VMEM((2 * L,), jnp.int32),
            pltpu.VMEM((NT * 2 * L,), jnp.int32),
            pltpu.VMEM_SHARED((2 * NT * 2 * L,), jnp.int32),
        ],
    )
    return call, npad, sl


def kernel(bbs, conf):
    n = conf.shape[0]
    call, npad, sl = _build(n)
    order = jnp.argsort(-conf)
    sb = bbs[order]
    sconf = conf[order]
    x1 = sb[:, 0]
    y1 = sb[:, 1]
    x2 = sb[:, 0] + sb[:, 2]
    y2 = sb[:, 1] + sb[:, 3]
    pad = npad - n

    def p(a):
        return jnp.concatenate([a, jnp.zeros((pad,), a.dtype)])

    conf_t = p(sconf).reshape(sl, NT).T
    out_t = call(p(x1), p(y1), p(x2), p(y2), conf_t)
    out_sorted = out_t.T.reshape(npad)
    return jnp.zeros_like(conf).at[order].set(out_sorted[:n])


# speedup vs baseline: 159.9927x; 1.6323x over previous
import functools

import jax
import jax.numpy as jnp
from jax import lax
from jax.experimental import pallas as pl
from jax.experimental.pallas import tpu as pltpu
from jax.experimental.pallas import tpu_sc as plsc

NMS_THR = 0.45
NT = 16
L = 16
CB = 64

def _sort16(v, lane):
    for k in (2, 4, 8, 16):
        j = k // 2
        while j >= 1:
            pv = v[jnp.bitwise_xor(lane, j)]
            takemin = ((lane & j) == 0) == ((lane & k) == 0)
            v = jnp.where(takemin, jnp.minimum(v, pv), jnp.maximum(v, pv))
            j //= 2
    return v


def _tec_body(x1h, y1h, x2h, y2h, confh, outh,
              vx1, vy1, vx2, vy2, outs, act, pubv, candv, kbuf, cand_sp,
              *, n, npad, sl):
    t = lax.axis_index("s")
    sent = jnp.int32(npad)
    lane = lax.iota(jnp.int32, L)
    zf = jnp.zeros((L,), jnp.float32)

    pltpu.sync_copy(x1h, vx1)
    pltpu.sync_copy(y1h, vy1)
    pltpu.sync_copy(x2h, vx2)
    pltpu.sync_copy(y2h, vy2)
    pltpu.sync_copy(confh.at[t], outs)

    def init_chunk(c, _):
        act[pl.ds(c * L, L)] = jnp.minimum((c * L + lane) * NT + t, sent - 1)
        return 0

    lax.fori_loop(0, sl // L, init_chunk, 0)
    cnt0 = (jnp.maximum(jnp.int32(n) - t, 0) + (NT - 1)) // NT

    def round_fn(cnt, par):
        v0 = act[pl.ds(0, L)]
        front = jnp.where(cnt > 0, v0[0], sent)
        second = jnp.where(cnt > 1, v0[1], sent)
        pubv[pl.ds(0, L)] = jnp.full((L,), front, jnp.int32)
        pubv[pl.ds(L, L)] = jnp.full((L,), second, jnp.int32)
        pltpu.sync_copy(pubv,
                        cand_sp.at[pl.ds((par * NT + t) * (2 * L), 2 * L)])
        plsc.subcore_barrier()
        pltpu.sync_copy(cand_sp.at[pl.ds(par * NT * 2 * L, NT * 2 * L)],
                        candv)

        fronts = jnp.full((L,), sent, jnp.int32)
        seconds = jnp.full((L,), sent, jnp.int32)
        for r in range(NT):
            sel = lane == r
            fronts = jnp.where(sel, candv[pl.ds(r * 2 * L, L)], fronts)
            seconds = jnp.where(sel, candv[pl.ds(r * 2 * L + L, L)], seconds)

        m = seconds
        for d in (8, 4, 2, 1):
            m = jnp.minimum(m, m[jnp.bitwise_xor(lane, d)])
        lim = m[0]
        limv = jnp.full((L,), lim, jnp.int32)

        cand = _sort16(fronts, lane)
        gdone = cand[0] >= sent
        cg = jnp.minimum(cand, sent - 1)
        cx1 = plsc.load_gather(vx1, [cg])
        cy1 = plsc.load_gather(vy1, [cg])
        cx2 = plsc.load_gather(vx2, [cg])
        cy2 = plsc.load_gather(vy2, [cg])
        carea = (jnp.maximum(cx2 - cx1, 0.0) * jnp.maximum(cy2 - cy1, 0.0))
        eligible = cand < limv
        ne = plsc.all_reduce_population_count(eligible)[0]

        def greedy_step(k, suppi):
            kf = jnp.full((L,), k, jnp.int32)
            kept_k = suppi[kf] == 0
            bx1 = cx1[kf]
            by1 = cy1[kf]
            bx2 = cx2[kf]
            by2 = cy2[kf]
            ba = carea[kf]
            xx1 = jnp.maximum(cx1, bx1)
            yy1 = jnp.maximum(cy1, by1)
            xx2 = jnp.minimum(cx2, bx2)
            yy2 = jnp.minimum(cy2, by2)
            inter = (jnp.maximum(xx2 - xx1, 0.0)
                     * jnp.maximum(yy2 - yy1, 0.0))
            iou = inter / (ba + carea - inter + 1e-12)
            hit = (iou > NMS_THR) & (lane > kf) & kept_k
            return suppi | hit.astype(jnp.int32)

        suppi = lax.fori_loop(0, ne, greedy_step, jnp.zeros((L,), jnp.int32))
        supp = suppi != 0
        keep = eligible & jnp.logical_not(supp)

        frontv = jnp.full((L,), front, jnp.int32)
        mydecided = front < lim
        mysupp_cnt = plsc.all_reduce_population_count(
            (cand == frontv) & supp & eligible)[0]
        myslot = jnp.minimum(front // NT, jnp.int32(sl - 1))
        plsc.store_scatter(outs, [jnp.full((L,), myslot, jnp.int32)], zf,
                           mask=(lane == 0) & mydecided & (mysupp_cnt > 0))

        nkept = plsc.all_reduce_population_count(keep)[0]
        plsc.store_compressed(kbuf.at[pl.ds(0, L)], cand, mask=keep)
        kidx = kbuf[pl.ds(0, L)]
        kmask = lane < jnp.full((L,), nkept, jnp.int32)
        kx1 = plsc.load_gather(vx1, [kidx], mask=kmask)
        ky1 = plsc.load_gather(vy1, [kidx], mask=kmask)
        kx2 = plsc.load_gather(vx2, [kidx], mask=kmask)
        ky2 = plsc.load_gather(vy2, [kidx], mask=kmask)
        karea = (jnp.maximum(kx2 - kx1, 0.0) * jnp.maximum(ky2 - ky1, 0.0))

        skip = jnp.where(mydecided, jnp.int32(1), jnp.int32(0))
        ncand = cnt - skip
        nchunks = (ncand + (L - 1)) // L

        def chunk(c, w):
            idxv = act[pl.ds(skip + c * L, L)]
            valid = (c * L + lane) < ncand
            gx1 = plsc.load_gather(vx1, [idxv], mask=valid)
            gy1 = plsc.load_gather(vy1, [idxv], mask=valid)
            gx2 = plsc.load_gather(vx2, [idxv], mask=valid)
            gy2 = plsc.load_gather(vy2, [idxv], mask=valid)
            ga = (jnp.maximum(gx2 - gx1, 0.0) * jnp.maximum(gy2 - gy1, 0.0))

            def kept_step(k, sacci):
                kf = jnp.full((L,), k, jnp.int32)
                bx1 = kx1[kf]
                by1 = ky1[kf]
                bx2 = kx2[kf]
                by2 = ky2[kf]
                ba = karea[kf]
                xx1 = jnp.maximum(gx1, bx1)
                yy1 = jnp.maximum(gy1, by1)
                xx2 = jnp.minimum(gx2, bx2)
                yy2 = jnp.minimum(gy2, by2)
                inter = (jnp.maximum(xx2 - xx1, 0.0)
                         * jnp.maximum(yy2 - yy1, 0.0))
                iou = inter / (ba + ga - inter + 1e-12)
                return sacci | (iou > NMS_THR).astype(jnp.int32)

            sacci = lax.fori_loop(0, nkept, kept_step,
                                  jnp.zeros((L,), jnp.int32))
            sacc = (sacci != 0) & valid
            keepv = valid & (sacci == 0)
            plsc.store_scatter(outs, [idxv // NT], zf, mask=sacc)
            plsc.store_compressed(act.at[pl.ds(w, L)], idxv, mask=keepv)
            return w + plsc.all_reduce_population_count(keepv)[0]

        newcnt = lax.fori_loop(0, nchunks, chunk, jnp.int32(0))
        return newcnt, gdone.astype(jnp.int32)

    def block_fn(b, carry):
        def run_block(carry):
            def one_round(i, carry):
                cnt, done = carry
                return lax.cond(done == 1, lambda c: (c, jnp.int32(1)),
                                lambda c: round_fn(c, i % 2), cnt)
            return lax.fori_loop(0, CB, one_round, carry)

        return lax.cond(carry[1] == 1, lambda c: c, run_block, carry)

    nblocks = (n + 1 + CB - 1) // CB
    lax.fori_loop(0, nblocks, block_fn, (cnt0, jnp.int32(0)))

    pltpu.sync_copy(outs, outh.at[t])


@functools.lru_cache(maxsize=None)
def _build(n):
    npad = -(-n // (NT * L)) * (NT * L)
    sl = npad // NT
    mesh = plsc.VectorSubcoreMesh(core_axis_name="c", subcore_axis_name="s",
                                  num_cores=1, num_subcores=NT)
    body = functools.partial(_tec_body, n=n, npad=npad, sl=sl)
    call = pl.kernel(
        body,
        out_type=jax.ShapeDtypeStruct((NT, sl), jnp.float32),
        mesh=mesh,
        compiler_params=pltpu.CompilerParams(needs_layout_passes=False),
        scratch_types=[
            pltpu.VMEM((npad,), jnp.float32),
            pltpu.VMEM((npad,), jnp.float32),
            pltpu.VMEM((npad,), jnp.float32),
            pltpu.VMEM((npad,), jnp.float32),
            pltpu.VMEM((sl,), jnp.float32),
            pltpu.VMEM((sl + L,), jnp.int32),
            pltpu.VMEM((2 * L,), jnp.int32),
            pltpu.VMEM((NT * 2 * L,), jnp.int32),
            pltpu.VMEM((L,), jnp.int32),
            pltpu.VMEM_SHARED((2 * NT * 2 * L,), jnp.int32),
        ],
    )
    return call, npad, sl


def kernel(bbs, conf):
    n = conf.shape[0]
    call, npad, sl = _build(n)
    order = jnp.argsort(-conf)
    sb = bbs[order]
    sconf = conf[order]
    x1 = sb[:, 0]
    y1 = sb[:, 1]
    x2 = sb[:, 0] + sb[:, 2]
    y2 = sb[:, 1] + sb[:, 3]
    pad = npad - n

    def p(a):
        return jnp.concatenate([a, jnp.zeros((pad,), a.dtype)])

    conf_t = p(sconf).reshape(sl, NT).T
    out_t = call(p(x1), p(y1), p(x2), p(y2), conf_t)
    out_sorted = out_t.T.reshape(npad)
    return jnp.zeros_like(conf).at[order].set(out_sorted[:n])
